# Initial kernel scaffold; baseline (speedup 1.0000x reference)
import jax
import jax.numpy as jnp
from jax.experimental import pallas as pl


def kernel(x, W_temp3, W_toqk, W_tov, W_out, b_out):
    raise NotImplementedError("write your pallas kernel here")


# speedup vs baseline: 3.4683x; 3.4683x over previous
import jax
import jax.numpy as jnp
from jax.experimental import pallas as pl

_T = 2048
_E = 768
_DIM = 1024
_HEADS = 16
_DH = 32
_DIM_HEADS = _HEADS * _DH
_OUP = 1024
_NEG = -5e4
_TBLK = 512


def _proj_kernel(x_ref, wt3_ref, wqk_ref, wv_ref, qk_ref, v_ref):
    h = jax.lax.dot_general(
        x_ref[...], wt3_ref[...], (((1,), (1,)), ((), ())),
        preferred_element_type=jnp.float32)
    qk_ref[...] = jax.lax.dot_general(
        h, wqk_ref[...], (((1,), (1,)), ((), ())),
        preferred_element_type=jnp.float32)
    v_ref[...] = jax.lax.dot_general(
        h, wv_ref[...], (((1,), (1,)), ((), ())),
        preferred_element_type=jnp.float32)


def _attn_kernel(qk_ref, v_ref, wo_ref, bout_ref, out_ref):
    head = pl.program_id(0)
    qk = qk_ref[0]
    v = v_ref[0]
    norm = jnp.sqrt(jnp.sum(qk * qk, axis=-1, keepdims=True))
    k = qk / jnp.maximum(norm, 1e-12)
    s = jax.lax.dot_general(
        qk, k, (((1,), (1,)), ((), ())),
        preferred_element_type=jnp.float32) * (_DH ** -0.5)
    rows = jax.lax.broadcasted_iota(jnp.int32, (_T, _T), 0)
    cols = jax.lax.broadcasted_iota(jnp.int32, (_T, _T), 1)
    s = jnp.where(rows == cols, _NEG, s)
    m = jnp.max(s, axis=-1, keepdims=True)
    e = jnp.exp(s - m)
    denom = jnp.sum(e, axis=-1, keepdims=True)
    o = jnp.dot(e, v, preferred_element_type=jnp.float32) / denom
    contrib = jax.lax.dot_general(
        o, wo_ref[0], (((1,), (1,)), ((), ())),
        preferred_element_type=jnp.float32)

    @pl.when(head == 0)
    def _():
        out_ref[...] = bout_ref[...] + contrib

    @pl.when(head != 0)
    def _():
        out_ref[...] += contrib


def kernel(x, W_temp3, W_toqk, W_tov, W_out, b_out):
    x2 = x[0]
    qk, v = pl.pallas_call(
        _proj_kernel,
        grid=(_T // _TBLK,),
        in_specs=[
            pl.BlockSpec((_TBLK, _E), lambda i: (i, 0)),
            pl.BlockSpec((_DIM, _E), lambda i: (0, 0)),
            pl.BlockSpec((_DIM_HEADS, _DIM), lambda i: (0, 0)),
            pl.BlockSpec((_DIM_HEADS, _DIM), lambda i: (0, 0)),
        ],
        out_specs=[
            pl.BlockSpec((_TBLK, _DIM_HEADS), lambda i: (i, 0)),
            pl.BlockSpec((_TBLK, _DIM_HEADS), lambda i: (i, 0)),
        ],
        out_shape=[
            jax.ShapeDtypeStruct((_T, _DIM_HEADS), jnp.float32),
            jax.ShapeDtypeStruct((_T, _DIM_HEADS), jnp.float32),
        ],
    )(x2, W_temp3, W_toqk, W_tov)

    qk3 = qk.reshape(_T, _HEADS, _DH).transpose(1, 0, 2)
    v3 = v.reshape(_T, _HEADS, _DH).transpose(1, 0, 2)
    wo3 = W_out.reshape(_OUP, _HEADS, _DH).transpose(1, 0, 2)

    out = pl.pallas_call(
        _attn_kernel,
        grid=(_HEADS,),
        in_specs=[
            pl.BlockSpec((1, _T, _DH), lambda h: (h, 0, 0)),
            pl.BlockSpec((1, _T, _DH), lambda h: (h, 0, 0)),
            pl.BlockSpec((1, _OUP, _DH), lambda h: (h, 0, 0)),
            pl.BlockSpec((1, _OUP), lambda h: (0, 0)),
        ],
        out_specs=pl.BlockSpec((_T, _OUP), lambda h: (0, 0)),
        out_shape=jax.ShapeDtypeStruct((_T, _OUP), jnp.float32),
    )(qk3, v3, wo3, b_out.reshape(1, _OUP))

    return out.reshape(1, _T, _OUP)


# speedup vs baseline: 4.3845x; 1.2642x over previous
import jax
import jax.numpy as jnp
from jax.experimental import pallas as pl

_T = 2048
_E = 768
_DIM = 1024
_HEADS = 16
_DH = 32
_DIM_HEADS = _HEADS * _DH
_OUP = 1024
_NEG = -5e4
_TBLK = 512


def _proj_kernel(x_ref, wt3_ref, wqk_ref, wv_ref, qk_ref, v_ref):
    h = jax.lax.dot_general(
        x_ref[...], wt3_ref[...], (((1,), (1,)), ((), ())),
        preferred_element_type=jnp.float32)
    qk_ref[...] = jax.lax.dot_general(
        h, wqk_ref[...], (((1,), (1,)), ((), ())),
        preferred_element_type=jnp.float32)
    v_ref[...] = jax.lax.dot_general(
        h, wv_ref[...], (((1,), (1,)), ((), ())),
        preferred_element_type=jnp.float32)


def _attn_kernel(qk_ref, va_ref, wo_ref, bout_ref, out_ref):
    head = pl.program_id(0)
    qk = qk_ref[0]
    va = va_ref[0]
    inv_sqrt = _DH ** -0.5
    norm = jnp.sqrt(jnp.sum(qk * qk, axis=-1, keepdims=True))
    k = qk / jnp.maximum(norm, 1e-12)
    s = jax.lax.dot_general(
        qk, k, (((1,), (1,)), ((), ())),
        preferred_element_type=jnp.float32) * inv_sqrt
    m = norm * inv_sqrt
    e = jnp.exp(s - m)
    num = jnp.dot(e, va, preferred_element_type=jnp.float32)
    denom = num[:, _DH:_DH + 1] - 1.0
    o = (num[:, :_DH] - va[:, :_DH]) / denom
    contrib = jax.lax.dot_general(
        o, wo_ref[0], (((1,), (1,)), ((), ())),
        preferred_element_type=jnp.float32)

    @pl.when(head == 0)
    def _():
        out_ref[...] = bout_ref[...] + contrib

    @pl.when(head != 0)
    def _():
        out_ref[...] += contrib


def kernel(x, W_temp3, W_toqk, W_tov, W_out, b_out):
    x2 = x[0]
    qk, v = pl.pallas_call(
        _proj_kernel,
        grid=(_T // _TBLK,),
        in_specs=[
            pl.BlockSpec((_TBLK, _E), lambda i: (i, 0)),
            pl.BlockSpec((_DIM, _E), lambda i: (0, 0)),
            pl.BlockSpec((_DIM_HEADS, _DIM), lambda i: (0, 0)),
            pl.BlockSpec((_DIM_HEADS, _DIM), lambda i: (0, 0)),
        ],
        out_specs=[
            pl.BlockSpec((_TBLK, _DIM_HEADS), lambda i: (i, 0)),
            pl.BlockSpec((_TBLK, _DIM_HEADS), lambda i: (i, 0)),
        ],
        out_shape=[
            jax.ShapeDtypeStruct((_T, _DIM_HEADS), jnp.float32),
            jax.ShapeDtypeStruct((_T, _DIM_HEADS), jnp.float32),
        ],
    )(x2, W_temp3, W_toqk, W_tov)

    qk3 = qk.reshape(_T, _HEADS, _DH).transpose(1, 0, 2)
    v3 = v.reshape(_T, _HEADS, _DH).transpose(1, 0, 2)
    ones = jnp.ones((_HEADS, _T, 1), jnp.float32)
    zeros = jnp.zeros((_HEADS, _T, _DH - 1), jnp.float32)
    va3 = jnp.concatenate([v3, ones, zeros], axis=-1)
    wo3 = W_out.reshape(_OUP, _HEADS, _DH).transpose(1, 0, 2)

    out = pl.pallas_call(
        _attn_kernel,
        grid=(_HEADS,),
        in_specs=[
            pl.BlockSpec((1, _T, _DH), lambda h: (h, 0, 0)),
            pl.BlockSpec((1, _T, 2 * _DH), lambda h: (h, 0, 0)),
            pl.BlockSpec((1, _OUP, _DH), lambda h: (h, 0, 0)),
            pl.BlockSpec((1, _OUP), lambda h: (0, 0)),
        ],
        out_specs=pl.BlockSpec((_T, _OUP), lambda h: (0, 0)),
        out_shape=jax.ShapeDtypeStruct((_T, _OUP), jnp.float32),
    )(qk3, va3, wo3, b_out.reshape(1, _OUP))

    return out.reshape(1, _T, _OUP)
